# Initial kernel scaffold; baseline (speedup 1.0000x reference)
import jax
import jax.numpy as jnp
from jax.experimental import pallas as pl


def kernel(node_feats, edge_index, W0, b0, W1, b1):
    raise NotImplementedError("write your pallas kernel here")


# speedup vs baseline: 32.7575x; 32.7575x over previous
import jax
import jax.numpy as jnp
from jax import lax
from jax.experimental import pallas as pl
from jax.experimental.pallas import tpu as pltpu
from jax.experimental.pallas import tpu_sc as plsc

N = 10000
E = 320000
D = 128
NC = 2
NS = 16
NW = NC * NS
EPW = E // NW
NPT = N // NS
N_PAD = 10240
SENT = N
HIT_CAP = EPW + NPT + 32
VEC = 16

import dataclasses
import functools


def _sc_params():
    cp = pltpu.CompilerParams()
    if "needs_layout_passes" in pltpu.CompilerParams.__dataclass_fields__:
        cp = dataclasses.replace(cp, needs_layout_passes=False)
    return cp


@functools.lru_cache(maxsize=1)
def _mesh():
    return plsc.VectorSubcoreMesh(core_axis_name="c", subcore_axis_name="s",
                                  num_cores=NC, num_subcores=NS)


def _hist_body(edges, hs_out, hd_out, hc_out, src_v, dst_v, hs, hd, hc, sem):
    cid = lax.axis_index("c")
    sid = lax.axis_index("s")
    wid = cid * NS + sid
    base = wid * EPW
    pltpu.async_copy(edges.at[pl.ds(base, EPW)], src_v, sem).wait()
    pltpu.async_copy(edges.at[pl.ds(E + base, EPW)], dst_v, sem).wait()

    zi = jnp.zeros((VEC,), jnp.int32)

    @pl.loop(0, N_PAD, step=VEC)
    def _(i):
        hs[pl.ds(i, VEC)] = zi
        hd[pl.ds(i, VEC)] = zi
        hc[pl.ds(i, VEC)] = zi

    @pl.loop(0, EPW, step=VEC)
    def _(i):
        s = src_v[pl.ds(i, VEC)]
        d = dst_v[pl.ds(i, VEC)]
        c1, m1 = plsc.scan_count(s)
        plsc.addupdate_scatter(hs, [s], c1, mask=m1)
        c2, m2 = plsc.scan_count(d)
        plsc.addupdate_scatter(hd, [d], c2, mask=m2)
        c3, m3 = plsc.scan_count(s, mask=d == 0)
        plsc.addupdate_scatter(hc, [s], c3, mask=m3)

    pltpu.sync_copy(hs, hs_out.at[pl.ds(wid * N_PAD, N_PAD)])
    pltpu.sync_copy(hd, hd_out.at[pl.ds(wid * N_PAD, N_PAD)])
    pltpu.sync_copy(hc, hc_out.at[pl.ds(wid * N_PAD, N_PAD)])


@functools.lru_cache(maxsize=1)
def _hist():
    return pl.kernel(
    _hist_body,
    out_type=tuple(jax.ShapeDtypeStruct((NW * N_PAD,), jnp.int32) for _ in range(3)),
    mesh=_mesh(),
    scratch_types=[
        pltpu.VMEM((EPW,), jnp.int32),
        pltpu.VMEM((EPW,), jnp.int32),
        pltpu.VMEM((N_PAD,), jnp.int32),
        pltpu.VMEM((N_PAD,), jnp.int32),
        pltpu.VMEM((N_PAD,), jnp.int32),
        pltpu.SemaphoreType.DMA,
    ],
    compiler_params=_sc_params(),
    )


def _norm_body(hs_ref, hd_ref, hc_ref, flag_ref, nsrc_ref, ndst_ref, w2_ref):
    cs = jnp.sum(hs_ref[...], axis=0)
    cd = jnp.sum(hd_ref[...], axis=0)
    cc = jnp.sum(hc_ref[...], axis=0)
    r = lax.broadcasted_iota(jnp.int32, (N_PAD // 128, 128), 0)
    c = lax.broadcasted_iota(jnp.int32, (N_PAD // 128, 128), 1)
    is0 = (r * 128 + c) == 0
    nsrc = lax.rsqrt((cs + 1).astype(jnp.float32))
    ndst = lax.rsqrt((cd + 1).astype(jnp.float32))
    flag_ref[...] = ((cc > 0) | is0).astype(jnp.int32)
    nsrc_ref[...] = nsrc
    ndst_ref[...] = ndst
    w2_ref[...] = (cc.astype(jnp.float32) + is0.astype(jnp.float32)) * nsrc * ndst[0:1, 0:1]


_norm = pl.pallas_call(
    _norm_body,
    out_shape=(
        jax.ShapeDtypeStruct((N_PAD // 128, 128), jnp.int32),
        jax.ShapeDtypeStruct((N_PAD // 128, 128), jnp.float32),
        jax.ShapeDtypeStruct((N_PAD // 128, 128), jnp.float32),
        jax.ShapeDtypeStruct((N_PAD // 128, 128), jnp.float32),
    ),
)


WIN = 2000
HIT_WCAP = WIN + VEC


def _agg_body(edges, x, flag_h, nsrc_h, zer_h, acc_out,
              src_v, dst_v, flag_v, nsrc_v, hit_s, hit_d, rows_v,
              acc_sh, sem):
    cid = lax.axis_index("c")
    sid = lax.axis_index("s")
    wid = cid * NS + sid
    base = wid * EPW
    pltpu.async_copy(flag_h, flag_v, sem).wait()
    pltpu.async_copy(nsrc_h, nsrc_v, sem).wait()
    pltpu.async_copy(zer_h, rows_v, sem).wait()
    iota = lax.iota(jnp.int32, VEC)
    vbase = sid * NPT
    nsweep = NS * ((NPT + VEC - 1) // VEC)

    @pl.loop(0, nsweep, step=VEC)
    def _(i):
        v = vbase + i + iota
        m = (plsc.load_gather(flag_v, [v]) != 0) & (i + iota < NPT)

        @pl.when(jnp.sum(m.astype(jnp.int32)) > 0)
        def _():
            idx = jnp.where(m, v, SENT)
            pltpu.sync_copy(rows_v, acc_sh.at[idx])

    plsc.subcore_barrier()

    def drain(cnt):
        nch = (cnt + VEC - 1) // VEC

        @pl.loop(0, nch)
        def _(j):
            off = j * VEC
            s16 = hit_s[pl.ds(off, VEC)]
            d16 = hit_d[pl.ds(off, VEC)]
            pltpu.async_copy(x.at[s16], rows_v, sem).wait()
            nv = plsc.load_gather(nsrc_v, [s16])

            @pl.loop(0, D)
            def _(f):
                fv = jnp.full((VEC,), f, jnp.int32)
                cv = plsc.load_gather(rows_v, [iota, fv])
                plsc.store_scatter(rows_v, [iota, fv], cv * nv)

            pltpu.sync_copy(rows_v, acc_sh.at[d16], add=True)

    @pl.loop(0, EPW, step=WIN)
    def _(w):
        pltpu.async_copy(edges.at[pl.ds(base + w, WIN)], src_v, sem).wait()
        pltpu.async_copy(edges.at[pl.ds(E + base + w, WIN)], dst_v, sem).wait()

        @pl.loop(0, WIN, step=VEC, init_carry=jnp.int32(0))
        def cnt_e(i, cnt):
            s = src_v[pl.ds(i, VEC)]
            d = dst_v[pl.ds(i, VEC)]
            m = plsc.load_gather(flag_v, [d]) != 0
            plsc.store_compressed(hit_s.at[pl.ds(cnt, VEC)], s, mask=m)
            plsc.store_compressed(hit_d.at[pl.ds(cnt, VEC)], d, mask=m)
            return cnt + jnp.sum(m.astype(jnp.int32))

        cnt = cnt_e
        hit_s[pl.ds(cnt, VEC)] = jnp.zeros((VEC,), jnp.int32)
        hit_d[pl.ds(cnt, VEC)] = jnp.full((VEC,), SENT, jnp.int32)
        drain(cnt)

    @pl.when(cid == 0)
    def _():
        @pl.loop(0, (NPT + VEC - 1) // VEC * VEC, step=VEC,
                 init_carry=jnp.int32(0))
        def cnt_s(i, cnt):
            v = vbase + i + iota
            m = (plsc.load_gather(flag_v, [v]) != 0) & (i + iota < NPT)
            plsc.store_compressed(hit_s.at[pl.ds(cnt, VEC)], v, mask=m)
            plsc.store_compressed(hit_d.at[pl.ds(cnt, VEC)], v, mask=m)
            return cnt + jnp.sum(m.astype(jnp.int32))

        cnt = cnt_s
        hit_s[pl.ds(cnt, VEC)] = jnp.zeros((VEC,), jnp.int32)
        hit_d[pl.ds(cnt, VEC)] = jnp.full((VEC,), SENT, jnp.int32)
        drain(cnt)

    plsc.subcore_barrier()

    rpt = N_PAD // NS
    pltpu.sync_copy(acc_sh.at[pl.ds(sid * rpt, rpt)],
                    acc_out.at[cid, pl.ds(sid * rpt, rpt)])


@functools.lru_cache(maxsize=1)
def _agg():
    return pl.kernel(
    _agg_body,
    out_type=jax.ShapeDtypeStruct((NC, N_PAD, D), jnp.float32),
    mesh=_mesh(),
    scratch_types=[
        pltpu.VMEM((WIN,), jnp.int32),
        pltpu.VMEM((WIN,), jnp.int32),
        pltpu.VMEM((N_PAD,), jnp.int32),
        pltpu.VMEM((N_PAD,), jnp.float32),
        pltpu.VMEM((HIT_WCAP,), jnp.int32),
        pltpu.VMEM((HIT_WCAP,), jnp.int32),
        pltpu.VMEM((VEC, D), jnp.float32),
        pltpu.VMEM_SHARED((N_PAD, D), jnp.float32),
        pltpu.SemaphoreType.DMA,
    ],
    compiler_params=_sc_params(),
    )


def _final_body(acc_ref, ndst_ref, flag_ref, w2_ref, w0_ref, b0_ref, w1_ref,
                b1_ref, out_ref):
    a = acc_ref[0] + acc_ref[1]
    a = a * ndst_ref[...]
    h = jnp.dot(a, w0_ref[...], precision=lax.Precision.HIGHEST,
                preferred_element_type=jnp.float32)
    h = jnp.maximum(h + b0_ref[...], 0.0)
    h = jnp.where(flag_ref[...] != 0, h, 0.0)
    v1 = jnp.dot(w2_ref[...], h, precision=lax.Precision.HIGHEST,
                 preferred_element_type=jnp.float32)
    v2 = jnp.maximum(jnp.dot(v1, w1_ref[...], precision=lax.Precision.HIGHEST,
                             preferred_element_type=jnp.float32) + b1_ref[...],
                     0.0)
    out_ref[...] = jnp.concatenate([v2, v2], axis=1)


_final = pl.pallas_call(
    _final_body,
    out_shape=jax.ShapeDtypeStruct((1, 2 * D), jnp.float32),
)


def kernel(node_feats, edge_index, W0, b0, W1, b1):
    edges_flat = edge_index.reshape(2 * E)
    hs, hd, hc = _hist()(edges_flat)
    flag, nsrc, ndst, w2 = _norm(
        hs.reshape(NW, N_PAD // 128, 128),
        hd.reshape(NW, N_PAD // 128, 128),
        hc.reshape(NW, N_PAD // 128, 128),
    )
    acc = _agg()(edges_flat, node_feats, flag.reshape(N_PAD),
               nsrc.reshape(N_PAD), jnp.zeros((VEC, D), jnp.float32))
    out = _final(acc, ndst.reshape(N_PAD, 1), flag.reshape(N_PAD, 1),
                 w2.reshape(1, N_PAD), W0, b0.reshape(1, D), W1,
                 b1.reshape(1, D))
    return out
